# baseline (speedup 1.0000x reference)
import functools

import jax
import jax.numpy as jnp
from jax import lax
from jax.experimental import pallas as pl
from jax.experimental.pallas import tpu as pltpu
from jax.experimental.pallas import tpu_sc as plsc

N_NODES = 10000
N_EDGES = 320000
D = 128
G = 64

NC = 2
NS = 16
NW = NC * NS

NPAD = 10240
EPAD = 327680
NCHUNKS = EPAD // 128
N0 = 80
N1 = 80
IBLK = 16
ROWS_PER_TILE = NPAD // NS

_SC_MESH = plsc.VectorSubcoreMesh(
    core_axis_name="c", subcore_axis_name="s", num_cores=NC, num_subcores=NS
)


def _chunk_range(c, s):
    n = jnp.where(c == 0, N0, N1)
    start = jnp.where(c == 0, s * N0, NS * N0 + s * N1)
    return start, n


def _deg_body(dst_hbm, zeros1_hbm, deg_out, idx_v, ones_v, deg_sh, sem):
    c = lax.axis_index("c")
    s = lax.axis_index("s")
    w = s * NC + c
    dchunks = NCHUNKS // NW

    pltpu.sync_copy(zeros1_hbm, deg_sh.at[pl.ds(s * ROWS_PER_TILE, ROWS_PER_TILE)])

    for k in range(8):
        ones_v[pl.ds(k * 16, 16)] = jnp.ones((16,), jnp.float32)

    pltpu.sync_copy(dst_hbm.at[pl.ds(w * dchunks, dchunks)], idx_v)
    plsc.subcore_barrier()

    def body(j, carry):
        pltpu.sync_copy(ones_v, deg_sh.at[idx_v.at[j]], add=True)
        return carry

    lax.fori_loop(0, dchunks, body, 0)
    plsc.subcore_barrier()

    pltpu.sync_copy(
        deg_sh.at[pl.ds(s * ROWS_PER_TILE, ROWS_PER_TILE)],
        deg_out.at[c, pl.ds(s * ROWS_PER_TILE, ROWS_PER_TILE)],
    )


_deg_call = pl.kernel(
    _deg_body,
    out_type=jax.ShapeDtypeStruct((NC, NPAD), jnp.float32),
    mesh=_SC_MESH,
    scratch_types=[
        pltpu.VMEM((NCHUNKS // NW, 128), jnp.int32),
        pltpu.VMEM((128,), jnp.float32),
        pltpu.VMEM_SHARED((NPAD,), jnp.float32),
        pltpu.SemaphoreType.DMA,
    ],
)


def _edge_body(p_hbm, src_hbm, dst_hbm, zeros2_hbm, acc_out,
               sidx_v, didx_v, rows_v, acc_sh, sem0, sem1, sem2, sem3):
    c = lax.axis_index("c")
    s = lax.axis_index("s")
    start, n = _chunk_range(c, s)

    pltpu.sync_copy(zeros2_hbm, acc_sh.at[pl.ds(s * ROWS_PER_TILE, ROWS_PER_TILE)])
    plsc.subcore_barrier()

    sems = (sem0, sem1, sem2, sem3)

    def block(bi, carry):
        pltpu.sync_copy(src_hbm.at[pl.ds(start + bi * IBLK, IBLK)], sidx_v)
        pltpu.sync_copy(dst_hbm.at[pl.ds(start + bi * IBLK, IBLK)], didx_v)

        for b in range(4):
            pltpu.async_copy(
                p_hbm.at[sidx_v.at[b // 2, pl.ds((b % 2) * 64, 64)]],
                rows_v.at[pl.ds(b * 64, 64)], sems[b])

        def group(g, c2):
            for q in range(2):
                j = g * 2 + q
                for half in range(2):
                    b = q * 2 + half
                    pltpu.make_async_copy(
                        p_hbm.at[sidx_v.at[0, pl.ds(0, 64)]],
                        rows_v.at[pl.ds(b * 64, 64)], sems[b]).wait()
                pltpu.sync_copy(rows_v.at[pl.ds(q * 128, 128)],
                                acc_sh.at[didx_v.at[j]], add=True)

                nxt = j + 2

                @pl.when(nxt < IBLK)
                def _():
                    for half in range(2):
                        b = q * 2 + half
                        pltpu.async_copy(
                            p_hbm.at[sidx_v.at[nxt, pl.ds(half * 64, 64)]],
                            rows_v.at[pl.ds(b * 64, 64)], sems[b])

            return c2

        lax.fori_loop(0, IBLK // 2, group, 0)
        return carry

    lax.fori_loop(0, n // IBLK, block, 0)
    plsc.subcore_barrier()

    pltpu.sync_copy(
        acc_sh.at[pl.ds(s * ROWS_PER_TILE, ROWS_PER_TILE)],
        acc_out.at[c, pl.ds(s * ROWS_PER_TILE, ROWS_PER_TILE)],
    )


_edge_call = pl.kernel(
    _edge_body,
    out_type=jax.ShapeDtypeStruct((NC, NPAD, D), jnp.float32),
    mesh=_SC_MESH,
    scratch_types=[
        pltpu.VMEM((IBLK, 128), jnp.int32),
        pltpu.VMEM((IBLK, 128), jnp.int32),
        pltpu.VMEM((256, D), jnp.float32),
        pltpu.VMEM_SHARED((NPAD, D), jnp.float32),
        pltpu.SemaphoreType.DMA,
        pltpu.SemaphoreType.DMA,
        pltpu.SemaphoreType.DMA,
        pltpu.SemaphoreType.DMA,
    ],
)


def _dinv_from(degT_ref):
    deg = 1.0 + degT_ref[:, 0:1] + degT_ref[:, 1:2]
    return lax.rsqrt(deg)


def _tc1_body(x_ref, w1_ref, degT_ref, p1_ref):
    dinv = _dinv_from(degT_ref)
    h = jnp.dot(x_ref[...], w1_ref[...], preferred_element_type=jnp.float32)
    p1_ref[...] = dinv * h


def _tc2_body(accp_ref, p1_ref, degT_ref, w2_ref, b1_ref, p2_ref):
    dinv = _dinv_from(degT_ref)
    acc = accp_ref[0] + accp_ref[1] + p1_ref[...]
    h1 = jnp.maximum(dinv * acc + b1_ref[...], 0.0)
    h2 = jnp.dot(h1, w2_ref[...], preferred_element_type=jnp.float32)
    p2_ref[...] = dinv * h2


def _tc3_body(accp_ref, p2_ref, degT_ref, b2_ref, batch_ref, wh_ref, bh_ref,
              out_ref):
    dinv = _dinv_from(degT_ref)
    acc = accp_ref[0] + accp_ref[1] + p2_ref[...]
    h2 = jnp.maximum(dinv * acc + b2_ref[...], 0.0)
    gids = lax.broadcasted_iota(jnp.int32, (G, NPAD), 0)
    m = (batch_ref[...] == gids).astype(jnp.float32)
    ssum = jnp.dot(m, h2, preferred_element_type=jnp.float32)
    cnt = jnp.sum(m, axis=1, keepdims=True)
    emb = ssum / jnp.maximum(cnt, 1.0)
    out_ref[...] = (
        jnp.dot(emb, wh_ref[...], preferred_element_type=jnp.float32)
        + bh_ref[...]
    )


_tc1_call = pl.pallas_call(
    _tc1_body, out_shape=jax.ShapeDtypeStruct((NPAD, D), jnp.float32)
)
_tc2_call = pl.pallas_call(
    _tc2_body, out_shape=jax.ShapeDtypeStruct((NPAD, D), jnp.float32)
)
_tc3_call = pl.pallas_call(
    _tc3_body, out_shape=jax.ShapeDtypeStruct((G, 2), jnp.float32)
)


@jax.jit
def kernel(x, edge_index, batch, W1, b1, W2, b2, Wh, bh):
    src = edge_index[0].astype(jnp.int32)
    dst = edge_index[1].astype(jnp.int32)
    batch = batch.astype(jnp.int32)

    pad_e = N_NODES + (jnp.arange(EPAD - N_EDGES, dtype=jnp.int32)
                       % (NPAD - N_NODES))
    src_w = jnp.concatenate([src, pad_e]).reshape(NCHUNKS, 128)
    dst_w = jnp.concatenate([dst, pad_e]).reshape(NCHUNKS, 128)

    x_pad = jnp.concatenate(
        [x, jnp.zeros((NPAD - N_NODES, D), jnp.float32)], axis=0
    )
    batch_pad = jnp.concatenate(
        [batch, jnp.full((NPAD - N_NODES,), G, jnp.int32)]
    ).reshape(1, NPAD)

    zeros1 = jnp.zeros((ROWS_PER_TILE,), jnp.float32)
    zeros2 = jnp.zeros((ROWS_PER_TILE, D), jnp.float32)

    degp = _deg_call(dst_w, zeros1)
    degT = jnp.transpose(degp)

    p1 = _tc1_call(x_pad, W1, degT)
    acc1 = _edge_call(p1, src_w, dst_w, zeros2)
    p2 = _tc2_call(acc1, p1, degT, W2, b1.reshape(1, D))
    acc2 = _edge_call(p2, src_w, dst_w, zeros2)
    logits = _tc3_call(
        acc2, p2, degT, b2.reshape(1, D), batch_pad, Wh, bh.reshape(1, 2)
    )
    return logits


# speedup vs baseline: 1.0664x; 1.0664x over previous
import functools

import jax
import jax.numpy as jnp
from jax import lax
from jax.experimental import pallas as pl
from jax.experimental.pallas import tpu as pltpu
from jax.experimental.pallas import tpu_sc as plsc

N_NODES = 10000
N_EDGES = 320000
D = 128
G = 64

NC = 2
NS = 16
NW = NC * NS

NPAD = 10240
EPAD = 327680
NCHUNKS = EPAD // 128
N0 = 80
N1 = 80
IBLK = 40
ROWS_PER_TILE = NPAD // NS

_SC_MESH = plsc.VectorSubcoreMesh(
    core_axis_name="c", subcore_axis_name="s", num_cores=NC, num_subcores=NS
)


def _chunk_range(c, s):
    n = jnp.where(c == 0, N0, N1)
    start = jnp.where(c == 0, s * N0, NS * N0 + s * N1)
    return start, n


def _deg_body(dst_hbm, zeros1_hbm, deg_out, idx_v, ones_v, deg_sh, sem):
    c = lax.axis_index("c")
    s = lax.axis_index("s")
    w = s * NC + c
    dchunks = NCHUNKS // NW

    pltpu.sync_copy(zeros1_hbm, deg_sh.at[pl.ds(s * ROWS_PER_TILE, ROWS_PER_TILE)])

    for k in range(8):
        ones_v[pl.ds(k * 16, 16)] = jnp.ones((16,), jnp.float32)

    pltpu.sync_copy(dst_hbm.at[pl.ds(w * dchunks, dchunks)], idx_v)
    plsc.subcore_barrier()

    def body(j, carry):
        pltpu.sync_copy(ones_v, deg_sh.at[idx_v.at[j]], add=True)
        return carry

    lax.fori_loop(0, dchunks, body, 0)
    plsc.subcore_barrier()

    pltpu.sync_copy(
        deg_sh.at[pl.ds(s * ROWS_PER_TILE, ROWS_PER_TILE)],
        deg_out.at[c, pl.ds(s * ROWS_PER_TILE, ROWS_PER_TILE)],
    )


_deg_call = pl.kernel(
    _deg_body,
    out_type=jax.ShapeDtypeStruct((NC, NPAD), jnp.float32),
    mesh=_SC_MESH,
    scratch_types=[
        pltpu.VMEM((NCHUNKS // NW, 128), jnp.int32),
        pltpu.VMEM((128,), jnp.float32),
        pltpu.VMEM_SHARED((NPAD,), jnp.float32),
        pltpu.SemaphoreType.DMA,
    ],
)


def _edge_body(p_hbm, src_hbm, dst_hbm, zeros2_hbm, acc_out,
               sidx_v, didx_v, rows0_v, rows1_v, acc_sh, sem0, sem1):
    c = lax.axis_index("c")
    s = lax.axis_index("s")
    start, n = _chunk_range(c, s)

    pltpu.sync_copy(zeros2_hbm, acc_sh.at[pl.ds(s * ROWS_PER_TILE, ROWS_PER_TILE)])
    plsc.subcore_barrier()

    rows = (rows0_v, rows1_v)
    sems = (sem0, sem1)

    def block(bi, carry):
        pltpu.sync_copy(src_hbm.at[pl.ds(start + bi * IBLK, IBLK)], sidx_v)
        pltpu.sync_copy(dst_hbm.at[pl.ds(start + bi * IBLK, IBLK)], didx_v)

        pltpu.async_copy(p_hbm.at[sidx_v.at[0]], rows0_v, sem0)
        pltpu.async_copy(p_hbm.at[sidx_v.at[1]], rows1_v, sem1)

        def group(g, c2):
            for b in range(2):
                j = g * 2 + b
                pltpu.make_async_copy(
                    p_hbm.at[sidx_v.at[0]], rows[b], sems[b]
                ).wait()
                pltpu.sync_copy(rows[b], acc_sh.at[didx_v.at[j]], add=True)

                nxt = j + 2

                @pl.when(nxt < IBLK)
                def _():
                    pltpu.async_copy(
                        p_hbm.at[sidx_v.at[nxt]], rows[b], sems[b]
                    )

            return c2

        lax.fori_loop(0, IBLK // 2, group, 0)
        return carry

    lax.fori_loop(0, n // IBLK, block, 0)
    plsc.subcore_barrier()

    pltpu.sync_copy(
        acc_sh.at[pl.ds(s * ROWS_PER_TILE, ROWS_PER_TILE)],
        acc_out.at[c, pl.ds(s * ROWS_PER_TILE, ROWS_PER_TILE)],
    )


_edge_call = pl.kernel(
    _edge_body,
    out_type=jax.ShapeDtypeStruct((NC, NPAD, D), jnp.float32),
    mesh=_SC_MESH,
    scratch_types=[
        pltpu.VMEM((IBLK, 128), jnp.int32),
        pltpu.VMEM((IBLK, 128), jnp.int32),
        pltpu.VMEM((128, D), jnp.float32),
        pltpu.VMEM((128, D), jnp.float32),
        pltpu.VMEM_SHARED((NPAD, D), jnp.float32),
        pltpu.SemaphoreType.DMA,
        pltpu.SemaphoreType.DMA,
    ],
)


def _dinv_from(degT_ref):
    deg = 1.0 + degT_ref[:, 0:1] + degT_ref[:, 1:2]
    return lax.rsqrt(deg)


def _tc1_body(x_ref, w1_ref, degT_ref, p1_ref):
    dinv = _dinv_from(degT_ref)
    h = jnp.dot(x_ref[...], w1_ref[...], preferred_element_type=jnp.float32)
    p1_ref[...] = dinv * h


def _tc2_body(accp_ref, p1_ref, degT_ref, w2_ref, b1_ref, p2_ref):
    dinv = _dinv_from(degT_ref)
    acc = accp_ref[0] + accp_ref[1] + p1_ref[...]
    h1 = jnp.maximum(dinv * acc + b1_ref[...], 0.0)
    h2 = jnp.dot(h1, w2_ref[...], preferred_element_type=jnp.float32)
    p2_ref[...] = dinv * h2


def _tc3_body(accp_ref, p2_ref, degT_ref, b2_ref, batch_ref, wh_ref, bh_ref,
              out_ref):
    dinv = _dinv_from(degT_ref)
    acc = accp_ref[0] + accp_ref[1] + p2_ref[...]
    h2 = jnp.maximum(dinv * acc + b2_ref[...], 0.0)
    gids = lax.broadcasted_iota(jnp.int32, (G, NPAD), 0)
    m = (batch_ref[...] == gids).astype(jnp.float32)
    ssum = jnp.dot(m, h2, preferred_element_type=jnp.float32)
    cnt = jnp.sum(m, axis=1, keepdims=True)
    emb = ssum / jnp.maximum(cnt, 1.0)
    out_ref[...] = (
        jnp.dot(emb, wh_ref[...], preferred_element_type=jnp.float32)
        + bh_ref[...]
    )


_tc1_call = pl.pallas_call(
    _tc1_body, out_shape=jax.ShapeDtypeStruct((NPAD, D), jnp.float32)
)
_tc2_call = pl.pallas_call(
    _tc2_body, out_shape=jax.ShapeDtypeStruct((NPAD, D), jnp.float32)
)
_tc3_call = pl.pallas_call(
    _tc3_body, out_shape=jax.ShapeDtypeStruct((G, 2), jnp.float32)
)


@jax.jit
def kernel(x, edge_index, batch, W1, b1, W2, b2, Wh, bh):
    src = edge_index[0].astype(jnp.int32)
    dst = edge_index[1].astype(jnp.int32)
    batch = batch.astype(jnp.int32)

    pad_e = N_NODES + (jnp.arange(EPAD - N_EDGES, dtype=jnp.int32)
                       % (NPAD - N_NODES))
    src_w = jnp.concatenate([src, pad_e]).reshape(NCHUNKS, 128)
    dst_w = jnp.concatenate([dst, pad_e]).reshape(NCHUNKS, 128)

    x_pad = jnp.concatenate(
        [x, jnp.zeros((NPAD - N_NODES, D), jnp.float32)], axis=0
    )
    batch_pad = jnp.concatenate(
        [batch, jnp.full((NPAD - N_NODES,), G, jnp.int32)]
    ).reshape(1, NPAD)

    zeros1 = jnp.zeros((ROWS_PER_TILE,), jnp.float32)
    zeros2 = jnp.zeros((ROWS_PER_TILE, D), jnp.float32)

    degp = _deg_call(dst_w, zeros1)
    degT = jnp.transpose(degp)

    p1 = _tc1_call(x_pad, W1, degT)
    acc1 = _edge_call(p1, src_w, dst_w, zeros2)
    p2 = _tc2_call(acc1, p1, degT, W2, b1.reshape(1, D))
    acc2 = _edge_call(p2, src_w, dst_w, zeros2)
    logits = _tc3_call(
        acc2, p2, degT, b2.reshape(1, D), batch_pad, Wh, bh.reshape(1, 2)
    )
    return logits
